# baseline (speedup 1.0000x reference)
import functools

import jax
import jax.numpy as jnp
from jax import lax
from jax.experimental import pallas as pl
from jax.experimental.pallas import tpu as pltpu
from jax.experimental.pallas import tpu_sc as plsc

N_NODES = 10000
HIDDEN = 128

NC = 2
NS = 16
NW = NC * NS

CHUNK = 128
CHUNKS_PER_TILE = 80
E_PER_TILE = CHUNK * CHUNKS_PER_TILE
E_PAD = E_PER_TILE * NW
N_PAD = 10240
ROWS_PER_TILE = N_PAD // NS

_mesh = plsc.VectorSubcoreMesh(
    core_axis_name="c", subcore_axis_name="s", num_cores=NC, num_subcores=NS
)


@functools.partial(
    pl.kernel,
    out_type=jax.ShapeDtypeStruct((NC, N_PAD, HIDDEN), jnp.float32),
    mesh=_mesh,
    scratch_types=dict(
        deg_sh=pltpu.VMEM_SHARED((N_PAD, HIDDEN), jnp.float32),
        col_v=pltpu.VMEM((CHUNKS_PER_TILE, CHUNK), jnp.int32),
        ones_v=pltpu.VMEM((CHUNK, HIDDEN), jnp.float32),
    ),
)
def _deg_kernel(col_hbm, ones_hbm, zeros_hbm, deg_out, *, deg_sh, col_v, ones_v):
    c = lax.axis_index("c")
    s = lax.axis_index("s")
    wid = c * NS + s
    pltpu.sync_copy(col_hbm.at[wid], col_v)
    pltpu.sync_copy(ones_hbm, ones_v)
    r0 = s * ROWS_PER_TILE
    pltpu.sync_copy(zeros_hbm.at[pl.ds(r0, ROWS_PER_TILE)],
                    deg_sh.at[pl.ds(r0, ROWS_PER_TILE)])
    plsc.subcore_barrier()
    for j in range(CHUNKS_PER_TILE):
        pltpu.sync_copy(ones_v, deg_sh.at[col_v.at[j]], add=True)
    plsc.subcore_barrier()
    pltpu.sync_copy(deg_sh.at[pl.ds(r0, ROWS_PER_TILE)],
                    deg_out.at[c, pl.ds(r0, ROWS_PER_TILE)])


@functools.partial(
    pl.kernel,
    out_type=jax.ShapeDtypeStruct((NC, N_PAD, HIDDEN), jnp.float32),
    mesh=_mesh,
    scratch_types=dict(
        acc_sh=pltpu.VMEM_SHARED((N_PAD, HIDDEN), jnp.float32),
        ridx_v=pltpu.VMEM((CHUNKS_PER_TILE // 2, CHUNK), jnp.int32),
        cidx_v=pltpu.VMEM((CHUNKS_PER_TILE // 2, CHUNK), jnp.int32),
        rows_a=pltpu.VMEM((CHUNK, HIDDEN), jnp.float32),
        rows_b=pltpu.VMEM((CHUNK, HIDDEN), jnp.float32),
        sem_a0=pltpu.SemaphoreType.DMA,
        sem_a1=pltpu.SemaphoreType.DMA,
        sem_b0=pltpu.SemaphoreType.DMA,
        sem_b1=pltpu.SemaphoreType.DMA,
    ),
)
def _scatter_kernel(row_hbm, col_hbm, xs_hbm, zeros_hbm, acc_out,
                    *, acc_sh, ridx_v, cidx_v, rows_a, rows_b,
                    sem_a0, sem_a1, sem_b0, sem_b1):
    c = lax.axis_index("c")
    s = lax.axis_index("s")
    wid = c * NS + s
    r0 = s * ROWS_PER_TILE
    pltpu.sync_copy(zeros_hbm.at[pl.ds(r0, ROWS_PER_TILE)],
                    acc_sh.at[pl.ds(r0, ROWS_PER_TILE)])
    plsc.subcore_barrier()
    bufs = (rows_a, rows_b)
    sems = ((sem_a0, sem_a1), (sem_b0, sem_b1))
    half = CHUNKS_PER_TILE // 2
    hc = CHUNK // 2

    def fire(j, b):
        return [pltpu.async_copy(
                    xs_hbm.at[ridx_v.at[j, pl.ds(k * hc, hc)]],
                    bufs[b].at[pl.ds(k * hc, hc)], sems[b][k])
                for k in range(2)]

    for h in range(2):
        pltpu.sync_copy(row_hbm.at[wid, pl.ds(h * half, half)], ridx_v)
        pltpu.sync_copy(col_hbm.at[wid, pl.ds(h * half, half)], cidx_v)
        descs = [fire(0, 0)]
        for j in range(half):
            for d in descs[j]:
                d.wait()
            if j + 1 < half:
                descs.append(fire(j + 1, (j + 1) % 2))
            pltpu.sync_copy(bufs[j % 2], acc_sh.at[cidx_v.at[j]], add=True)
    plsc.subcore_barrier()
    pltpu.sync_copy(acc_sh.at[pl.ds(r0, ROWS_PER_TILE)],
                    acc_out.at[c, pl.ds(r0, ROWS_PER_TILE)])


def _xs_body(emb_ref, w_ref, dega_ref, degb_ref, xs_ref, dis_ref):
    deg = dega_ref[0, :, 0:1] + degb_ref[0, :, 0:1] + 1.0
    dis = lax.rsqrt(deg)
    x = jnp.dot(emb_ref[...], w_ref[...], preferred_element_type=jnp.float32,
                precision=lax.Precision.HIGHEST)
    xs_ref[...] = x * dis
    dis_ref[...] = dis


def _combine_body(emb_ref, xs_ref, acca_ref, accb_ref, dis_ref, b_ref,
                  alpha_ref, out_ref):
    alpha = alpha_ref[0, 0]
    comb = xs_ref[...] + acca_ref[0] + accb_ref[0]
    gcn = dis_ref[...] * comb + b_ref[...]
    out_ref[...] = alpha * emb_ref[...] + (1.0 - alpha) * gcn


_BLK = 400


def kernel(drug_emb, ddi_edge_index, W, b, alpha):
    n = N_NODES
    row = ddi_edge_index[0].astype(jnp.int32)
    col = ddi_edge_index[1].astype(jnp.int32)
    e = row.shape[0]
    e_per_tile_real = e // NW
    pad_per_tile = E_PER_TILE - e_per_tile_real
    row3 = jnp.pad(row.reshape(NW, e_per_tile_real), ((0, 0), (0, pad_per_tile)),
                   constant_values=0).reshape(NW, CHUNKS_PER_TILE, CHUNK)
    col3 = jnp.pad(col.reshape(NW, e_per_tile_real), ((0, 0), (0, pad_per_tile)),
                   constant_values=n).reshape(NW, CHUNKS_PER_TILE, CHUNK)

    onesH = jnp.ones((CHUNK, HIDDEN), jnp.float32)
    zerosH = jnp.zeros((N_PAD, HIDDEN), jnp.float32)

    deg = _deg_kernel(col3, onesH, zerosH)

    grid = n // _BLK
    xs, dis = pl.pallas_call(
        _xs_body,
        grid=(grid,),
        in_specs=[
            pl.BlockSpec((_BLK, HIDDEN), lambda j: (j, 0)),
            pl.BlockSpec((HIDDEN, HIDDEN), lambda j: (0, 0)),
            pl.BlockSpec((1, _BLK, HIDDEN), lambda j: (0, j, 0)),
            pl.BlockSpec((1, _BLK, HIDDEN), lambda j: (1, j, 0)),
        ],
        out_specs=[
            pl.BlockSpec((_BLK, HIDDEN), lambda j: (j, 0)),
            pl.BlockSpec((_BLK, 1), lambda j: (j, 0)),
        ],
        out_shape=[
            jax.ShapeDtypeStruct((n, HIDDEN), jnp.float32),
            jax.ShapeDtypeStruct((n, 1), jnp.float32),
        ],
    )(drug_emb, W, deg, deg)

    acc = _scatter_kernel(row3, col3, xs, zerosH)

    enhanced = pl.pallas_call(
        _combine_body,
        grid=(grid,),
        in_specs=[
            pl.BlockSpec((_BLK, HIDDEN), lambda j: (j, 0)),
            pl.BlockSpec((_BLK, HIDDEN), lambda j: (j, 0)),
            pl.BlockSpec((1, _BLK, HIDDEN), lambda j: (0, j, 0)),
            pl.BlockSpec((1, _BLK, HIDDEN), lambda j: (1, j, 0)),
            pl.BlockSpec((_BLK, 1), lambda j: (j, 0)),
            pl.BlockSpec((1, HIDDEN), lambda j: (0, 0)),
            pl.BlockSpec((1, 1), lambda j: (0, 0)),
        ],
        out_specs=pl.BlockSpec((_BLK, HIDDEN), lambda j: (j, 0)),
        out_shape=jax.ShapeDtypeStruct((n, HIDDEN), jnp.float32),
    )(drug_emb, xs, acc, acc, dis, b.reshape(1, HIDDEN),
      alpha.reshape(1, 1))

    return enhanced


# speedup vs baseline: 1.2819x; 1.2819x over previous
import functools

import jax
import jax.numpy as jnp
from jax import lax
from jax.experimental import pallas as pl
from jax.experimental.pallas import tpu as pltpu
from jax.experimental.pallas import tpu_sc as plsc

N_NODES = 10000
HIDDEN = 128

NC = 2
NS = 16
NW = NC * NS

CHUNK = 128
CHUNKS_PER_TILE = 80
E_PER_TILE = CHUNK * CHUNKS_PER_TILE
E_PAD = E_PER_TILE * NW
N_PAD = 10240
ROWS_PER_TILE = N_PAD // NS

_mesh = plsc.VectorSubcoreMesh(
    core_axis_name="c", subcore_axis_name="s", num_cores=NC, num_subcores=NS
)


@functools.partial(
    pl.kernel,
    out_type=jax.ShapeDtypeStruct((NC, N_PAD, HIDDEN), jnp.float32),
    mesh=_mesh,
    scratch_types=dict(
        deg_sh=pltpu.VMEM_SHARED((N_PAD, HIDDEN), jnp.float32),
        col_v=pltpu.VMEM((CHUNKS_PER_TILE, CHUNK), jnp.int32),
        ones_v=pltpu.VMEM((CHUNK, HIDDEN), jnp.float32),
    ),
)
def _deg_kernel(col_hbm, ones_hbm, zeros_hbm, deg_out, *, deg_sh, col_v, ones_v):
    c = lax.axis_index("c")
    s = lax.axis_index("s")
    wid = c * NS + s
    pltpu.sync_copy(col_hbm.at[wid], col_v)
    pltpu.sync_copy(ones_hbm, ones_v)
    r0 = s * ROWS_PER_TILE
    pltpu.sync_copy(zeros_hbm.at[pl.ds(r0, ROWS_PER_TILE)],
                    deg_sh.at[pl.ds(r0, ROWS_PER_TILE)])
    plsc.subcore_barrier()
    for j in range(CHUNKS_PER_TILE):
        pltpu.sync_copy(ones_v, deg_sh.at[col_v.at[j]], add=True)
    plsc.subcore_barrier()
    pltpu.sync_copy(deg_sh.at[pl.ds(r0, ROWS_PER_TILE)],
                    deg_out.at[c, pl.ds(r0, ROWS_PER_TILE)])


N_HALF = 5120
ACC_ROWS = N_HALF + 8
IDX_ROWS = 160
SLABS = IDX_ROWS // 8
CSZ = (24, 24, 24, 24, 24, 8)
COFF = (0, 24, 48, 72, 96, 120)


@functools.partial(
    pl.kernel,
    out_type=jax.ShapeDtypeStruct((NC, N_HALF, HIDDEN), jnp.float32),
    mesh=_mesh,
    scratch_types=dict(
        xs_sh=pltpu.VMEM_SHARED((N_NODES, HIDDEN), jnp.float32),
        acc_sh=pltpu.VMEM_SHARED((ACC_ROWS, HIDDEN), jnp.float32),
        pk_v=pltpu.VMEM((2, 8, CHUNK), jnp.int32),
        rstage=pltpu.VMEM((1, CHUNK), jnp.int32),
        cstage=pltpu.VMEM((1, CHUNK), jnp.int32),
        rows_a=pltpu.VMEM((24, HIDDEN), jnp.float32),
        rows_b=pltpu.VMEM((24, HIDDEN), jnp.float32),
        sem_s0=pltpu.SemaphoreType.DMA,
        sem_s1=pltpu.SemaphoreType.DMA,
        sem_g0=pltpu.SemaphoreType.DMA,
        sem_g1=pltpu.SemaphoreType.DMA,
    ),
)
def _scatter_kernel(pk_hbm, xs_hbm, zeros_hbm, acc_out,
                    *, xs_sh, acc_sh, pk_v, rstage, cstage, rows_a, rows_b,
                    sem_s0, sem_s1, sem_g0, sem_g1):
    c = lax.axis_index("c")
    s = lax.axis_index("s")
    base = c * N_HALF
    @pl.when(s < 9)
    def _():
        pltpu.sync_copy(xs_hbm.at[pl.ds(s * 1024, 1024)],
                        xs_sh.at[pl.ds(s * 1024, 1024)])
    @pl.when(s == 9)
    def _():
        pltpu.sync_copy(xs_hbm.at[pl.ds(9216, 784)],
                        xs_sh.at[pl.ds(9216, 784)])
    pltpu.sync_copy(zeros_hbm.at[pl.ds(s * (N_HALF // NS), N_HALF // NS)],
                    acc_sh.at[pl.ds(s * (N_HALF // NS), N_HALF // NS)])
    plsc.subcore_barrier()

    bufs = (rows_a, rows_b)
    gsems = (sem_g0, sem_g1)
    ssems = (sem_s0, sem_s1)

    def process_slab(p):
        for r in range(8):
            for k in range(8):
                v = pk_v[p, r, pl.ds(k * 16, 16)]
                rstage[0, pl.ds(k * 16, 16)] = lax.bitwise_and(v, 0xFFFF)
                t = lax.shift_right_logical(v, 16) - base
                ok = jnp.logical_and(t >= 0, t < N_HALF)
                cstage[0, pl.ds(k * 16, 16)] = jnp.where(
                    ok, t, N_HALF + lax.bitwise_and(t, 7))
            descs = [pltpu.async_copy(
                xs_sh.at[rstage.at[0, pl.ds(COFF[0], CSZ[0])]],
                bufs[0].at[pl.ds(0, CSZ[0])], gsems[0])]
            for k in range(6):
                descs[k].wait()
                if k + 1 < 6:
                    descs.append(pltpu.async_copy(
                        xs_sh.at[rstage.at[0, pl.ds(COFF[k + 1], CSZ[k + 1])]],
                        bufs[(k + 1) % 2].at[pl.ds(0, CSZ[k + 1])],
                        gsems[(k + 1) % 2]))
                pltpu.sync_copy(
                    bufs[k % 2].at[pl.ds(0, CSZ[k])],
                    acc_sh.at[cstage.at[0, pl.ds(COFF[k], CSZ[k])]],
                    add=True)

    def slab_dma(i, p):
        off = pl.multiple_of(i * 8, 8)
        return pltpu.async_copy(pk_hbm.at[s, pl.ds(off, 8)], pk_v.at[p],
                                ssems[p])

    d0 = slab_dma(0, 0)

    def body(i2, _):
        a = i2 * 2
        d0.wait()
        db = slab_dma(a + 1, 1)
        process_slab(0)
        db.wait()
        @pl.when(i2 < SLABS // 2 - 1)
        def _():
            slab_dma(a + 2, 0)
        process_slab(1)
        return ()

    lax.fori_loop(0, SLABS // 2, body, ())
    plsc.subcore_barrier()
    r0 = s * (N_HALF // NS)
    pltpu.sync_copy(acc_sh.at[pl.ds(r0, N_HALF // NS)],
                    acc_out.at[c, pl.ds(r0, N_HALF // NS)])


def _xs_body(emb_ref, w_ref, dega_ref, degb_ref, xs_ref, dis_ref):
    deg = dega_ref[0, :, 0:1] + degb_ref[0, :, 0:1] + 1.0
    dis = lax.rsqrt(deg)
    x = jnp.dot(emb_ref[...], w_ref[...], preferred_element_type=jnp.float32,
                precision=lax.Precision.HIGHEST)
    xs_ref[...] = x * dis
    dis_ref[...] = dis


def _combine_body(emb_ref, xs_ref, acc_ref, dis_ref, b_ref,
                  alpha_ref, out_ref):
    alpha = alpha_ref[0, 0]
    comb = xs_ref[...] + acc_ref[...]
    gcn = dis_ref[...] * comb + b_ref[...]
    out_ref[...] = alpha * emb_ref[...] + (1.0 - alpha) * gcn


_BLK = 400


def kernel(drug_emb, ddi_edge_index, W, b, alpha):
    n = N_NODES
    row = ddi_edge_index[0].astype(jnp.int32)
    col = ddi_edge_index[1].astype(jnp.int32)
    e = row.shape[0]
    e_per_tile_real = e // NW
    pad_per_tile = E_PER_TILE - e_per_tile_real
    col3 = jnp.pad(col.reshape(NW, e_per_tile_real), ((0, 0), (0, pad_per_tile)),
                   constant_values=n).reshape(NW, CHUNKS_PER_TILE, CHUNK)
    packed = jnp.bitwise_or(row, jnp.left_shift(col, 16))
    epp_real = e // NS
    pk3 = jnp.pad(packed.reshape(NS, epp_real),
                  ((0, 0), (0, NS * IDX_ROWS * CHUNK // NS - epp_real)),
                  constant_values=n << 16).reshape(NS, IDX_ROWS, CHUNK)

    onesH = jnp.ones((CHUNK, HIDDEN), jnp.float32)
    zerosH = jnp.zeros((N_PAD, HIDDEN), jnp.float32)

    deg = _deg_kernel(col3, onesH, zerosH)

    grid = n // _BLK
    xs, dis = pl.pallas_call(
        _xs_body,
        grid=(grid,),
        in_specs=[
            pl.BlockSpec((_BLK, HIDDEN), lambda j: (j, 0)),
            pl.BlockSpec((HIDDEN, HIDDEN), lambda j: (0, 0)),
            pl.BlockSpec((1, _BLK, HIDDEN), lambda j: (0, j, 0)),
            pl.BlockSpec((1, _BLK, HIDDEN), lambda j: (1, j, 0)),
        ],
        out_specs=[
            pl.BlockSpec((_BLK, HIDDEN), lambda j: (j, 0)),
            pl.BlockSpec((_BLK, 1), lambda j: (j, 0)),
        ],
        out_shape=[
            jax.ShapeDtypeStruct((n, HIDDEN), jnp.float32),
            jax.ShapeDtypeStruct((n, 1), jnp.float32),
        ],
    )(drug_emb, W, deg, deg)

    acc = _scatter_kernel(pk3, xs, zerosH)
    acc_flat = acc.reshape(NC * N_HALF, HIDDEN)

    enhanced = pl.pallas_call(
        _combine_body,
        grid=(grid,),
        in_specs=[
            pl.BlockSpec((_BLK, HIDDEN), lambda j: (j, 0)),
            pl.BlockSpec((_BLK, HIDDEN), lambda j: (j, 0)),
            pl.BlockSpec((_BLK, HIDDEN), lambda j: (j, 0)),
            pl.BlockSpec((_BLK, 1), lambda j: (j, 0)),
            pl.BlockSpec((1, HIDDEN), lambda j: (0, 0)),
            pl.BlockSpec((1, 1), lambda j: (0, 0)),
        ],
        out_specs=pl.BlockSpec((_BLK, HIDDEN), lambda j: (j, 0)),
        out_shape=jax.ShapeDtypeStruct((n, HIDDEN), jnp.float32),
    )(drug_emb, xs, acc_flat, dis, b.reshape(1, HIDDEN),
      alpha.reshape(1, 1))

    return enhanced
